# baseline (speedup 1.0000x reference)
import jax
import jax.numpy as jnp
from jax import lax
from jax.experimental import pallas as pl
from jax.experimental.pallas import tpu as pltpu
from jax.experimental.pallas import tpu_sc as plsc

_NUMEL = 16777216
_NNZ = 1677721
_LR = 0.1

_NC = 2
_NS = 16
_NW = _NC * _NS
_G = 1024
_GROUPS = 52
_P = _G * _GROUPS
_TOTAL = _NW * _P


def _sc_body(idx_hbm, val_hbm, param_hbm, out_ref,
             idx_v, val_v, gat_v, sem_ld, sem_g, sem_s):
    c = lax.axis_index("c")
    s = lax.axis_index("s")
    wid = s * _NC + c
    base0 = wid * _P

    @pl.loop(0, _GROUPS)
    def _grp(g):
        off = base0 + g * _G
        ld_i = pltpu.make_async_copy(idx_hbm.at[pl.ds(off, _G)], idx_v, sem_ld)
        ld_v = pltpu.make_async_copy(val_hbm.at[pl.ds(off, _G)], val_v, sem_ld)
        ld_i.start()
        ld_v.start()
        ld_i.wait()
        ld_v.wait()
        gat = pltpu.make_async_copy(param_hbm.at[idx_v], gat_v, sem_g)
        gat.start()
        gat.wait()

        @pl.loop(0, _G // 16)
        def _cmp(i):
            sl = pl.ds(i * 16, 16)
            gat_v[sl] = gat_v[sl] + _LR * val_v[sl]

        sc = pltpu.make_async_copy(gat_v, out_ref.at[idx_v], sem_s)
        sc.start()
        sc.wait()


_sc_update = pl.kernel(
    _sc_body,
    out_type=(),
    mesh=plsc.VectorSubcoreMesh(core_axis_name="c", subcore_axis_name="s"),
    scratch_types=[
        pltpu.VMEM((_G,), jnp.int32),
        pltpu.VMEM((_G,), jnp.float32),
        pltpu.VMEM((_G,), jnp.float32),
        pltpu.SemaphoreType.DMA,
        pltpu.SemaphoreType.DMA,
        pltpu.SemaphoreType.DMA,
    ],
)


def kernel(param, values, indices):
    idx = indices.astype(jnp.int32)
    pad = _TOTAL - _NNZ
    idxp = jnp.pad(idx, (0, pad), mode="wrap")
    valp = jnp.pad(values, (0, pad), mode="wrap")
    out_ref = jax.new_ref(param)
    _sc_update(idxp, valp, param, out_ref)
    return out_ref[...]
